# baseline (speedup 1.0000x reference)
import functools

import jax
import jax.numpy as jnp
from jax import lax
from jax.experimental import pallas as pl
from jax.experimental.pallas import tpu as pltpu
from jax.experimental.pallas import tpu_sc as plsc

F = 26
V = 100000
D = 16
B = 16384

NC = 2
NS = 16
NW = NC * NS
LANES = 16

R = F * D
RPW = R // NW
OCH = 4096
NOC = B // OCH


def _sc_rowgather(idx_t, tab_t):
    mesh = plsc.VectorSubcoreMesh(core_axis_name="c", subcore_axis_name="s")

    @functools.partial(
        pl.kernel,
        mesh=mesh,
        compiler_params=pltpu.CompilerParams(
            use_tc_tiling_on_sc=True, needs_layout_passes=False
        ),
        out_type=jax.ShapeDtypeStruct((R, B), jnp.float32),
        scratch_types=[
            pltpu.VMEM((V,), jnp.float32),
            pltpu.VMEM((B,), jnp.int32),
            pltpu.VMEM((2, OCH), jnp.float32),
            pltpu.SemaphoreType.DMA,
            pltpu.SemaphoreType.DMA,
        ],
    )
    def k(idx_hbm, tab_hbm, out_hbm, row_v, idx_v, obuf, osem, rsem):
        wid = lax.axis_index("s") * NC + lax.axis_index("c")
        base = wid * RPW

        waits = []
        prev_fld = jnp.int32(-1)
        for k_row in range(RPW):
            r = base + k_row
            fld = lax.div(r, D)
            rowcp = pltpu.async_copy(tab_hbm.at[r], row_v, rsem)

            @pl.when(fld != prev_fld)
            def _():
                pltpu.sync_copy(idx_hbm.at[fld], idx_v)

            prev_fld = fld
            rowcp.wait()

            for c in range(NOC):
                bsel = c % 2
                if len(waits) >= 2:
                    waits.pop(0).wait()

                @plsc.parallel_loop(0, OCH // LANES, unroll=8)
                def _(j):
                    sl = pl.ds(pl.multiple_of(c * OCH + j * LANES, 8), LANES)
                    iv = idx_v[sl]
                    vals = plsc.load_gather(row_v, [iv])
                    obuf[bsel, pl.ds(pl.multiple_of(j * LANES, 8), LANES)] = vals

                waits.append(
                    pltpu.async_copy(
                        obuf.at[bsel], out_hbm.at[r, pl.ds(c * OCH, OCH)], osem
                    )
                )
        for w in waits:
            w.wait()

    return k(idx_t, tab_t)


def kernel(indices, tables):
    idx_t = indices.T
    tab_t = jnp.transpose(tables, (0, 2, 1)).reshape(R, V)
    out_t = _sc_rowgather(idx_t, tab_t)
    return out_t.T.reshape(B, F * D)
